# baseline (speedup 1.0000x reference)
import math

import jax
import jax.numpy as jnp
from jax import lax
from jax.experimental import pallas as pl
from jax.experimental.pallas import tpu as pltpu
from jax.experimental.pallas import tpu_sc as plsc

_V = 1_000_000
_D = 64
_L = 16
_NC, _NS = 2, 16
_NW = _NC * _NS
_B, _S = 4096, 200
_CB = _B // 128
_SCALE = math.sqrt(_D)

_MESH = plsc.VectorSubcoreMesh(core_axis_name="c", subcore_axis_name="s")


_AW = 256
_AU = (_V // _AW)
_AV0MAX = (_AU - 1) * _AW
_AT0 = _AU * _AW
_ATW = _V - _AT0
_AK = 124


def _repack_body(tabT, out_lin, inb0, inb1, outb0, outb1, tin, tout,
                 si0, si1, so0, so1):
    inb = (inb0, inb1)
    outb = (outb0, outb1)
    sem_in = (si0, si1)
    sem_out = (so0, so1)
    wid = lax.axis_index("s") * _NC + lax.axis_index("c")
    iota_row = lax.iota(jnp.int32, _L) * _D

    def v0_of(k):
        return jnp.minimum((k * _NW + wid) * _AW, _AV0MAX)

    def rd(k, b):
        return pltpu.make_async_copy(
            tabT.at[:, pl.ds(v0_of(k), _AW)], inb[b], sem_in[b])

    def wr(k, b):
        return pltpu.make_async_copy(
            outb[b], out_lin.at[pl.ds(v0_of(k) * _D, _AW * _D)], sem_out[b])

    def transpose(b):
        def row(f, carry):
            for m in range(_AW // _L):
                vec = inb[b][f, pl.ds(m * _L, _L)] * _SCALE
                plsc.store_scatter(
                    outb[b], [iota_row + (m * _L * _D + f)], vec)
            return carry
        lax.fori_loop(0, _D, row, 0)

    for b in range(2):
        rd(b, b).start()

    @pl.when(wid == 0)
    def _tail():
        pltpu.sync_copy(tabT.at[:, pl.ds(_AT0, _ATW)], tin)

        def row(f, carry):
            for m in range(_ATW // _L):
                vec = tin[f, pl.ds(m * _L, _L)] * _SCALE
                plsc.store_scatter(
                    tout, [iota_row + (m * _L * _D + f)], vec)
            return carry
        lax.fori_loop(0, _D, row, 0)
        pltpu.sync_copy(tout, out_lin.at[pl.ds(_AT0 * _D, _ATW * _D)])

    for b in range(2):
        rd(b, b).wait()
        transpose(b)
        wr(b, b).start()
        rd(b + 2, b).start()

    def pair(t, carry):
        for b in range(2):
            k = t * 2 + b
            rd(k, b).wait()
            wr(k, b).wait()
            transpose(b)
            wr(k, b).start()
            rd(k + 2, b).start()
        return carry

    lax.fori_loop(1, _AK // 2 - 1, pair, 0)

    for b in range(2):
        k = _AK - 2 + b
        rd(k, b).wait()
        wr(k, b).wait()
        transpose(b)
        wr(k, b).start()

    for b in range(2):
        wr(0, b).wait()


_repack = pl.kernel(
    _repack_body,
    mesh=_MESH,
    out_type=jax.ShapeDtypeStruct((_V * _D,), jnp.float32),
    scratch_types=[
        pltpu.VMEM((_D, _AW), jnp.float32),
        pltpu.VMEM((_D, _AW), jnp.float32),
        pltpu.VMEM((_AW * _D,), jnp.float32),
        pltpu.VMEM((_AW * _D,), jnp.float32),
        pltpu.VMEM((_D, _ATW), jnp.float32),
        pltpu.VMEM((_ATW * _D,), jnp.float32),
    ] + [pltpu.SemaphoreType.DMA] * 4,
    compiler_params=pltpu.CompilerParams(needs_layout_passes=False),
)


_NBUF = 4


def _lookup_body(tokT, table2d, out5, idx_v, inb, outb, *sems):
    sem_in = sems[:_NBUF]
    sem_out = sems[_NBUF:]
    wid = lax.axis_index("s") * _NC + lax.axis_index("c")
    iota = lax.iota(jnp.int32, _L)
    dsub = []
    for j in range(_D // _L):
        dvec = iota + j * _L
        dsub.append((lax.shift_right_logical(dvec, 3), dvec & 7))

    pltpu.sync_copy(tokT.at[:, pl.ds(wid * 128, 128)], idx_v)

    def gather(s, b):
        return pltpu.make_async_copy(
            table2d.at[idx_v.at[s]], inb.at[b], sem_in[b])

    def put(s, b):
        return pltpu.make_async_copy(
            outb.at[b], out5.at[s, :, wid], sem_out[b])

    def transpose(b):
        for j in range(_D // _L):
            fb, fi = dsub[j]

            def col(t, carry, j=j, fb=fb, fi=fi):
                vec = inb[b, t, pl.ds(j * _L, _L)]
                plsc.store_scatter(
                    outb.at[b], [fb, fi, jnp.full((_L,), 0, jnp.int32) + t],
                    vec)
                return carry

            lax.fori_loop(0, 128, col, 0)

    for b in range(_NBUF):
        gather(b, b).start()

    for b in range(_NBUF):
        gather(b, b).wait()
        transpose(b)
        put(b, b).start()
        gather(_NBUF + b, b).start()

    def ring(t, carry):
        for b in range(_NBUF):
            s = t * _NBUF + b
            gather(s, b).wait()
            put(s, b).wait()
            transpose(b)
            put(s, b).start()
            gather(s + _NBUF, b).start()
        return carry

    lax.fori_loop(1, _S // _NBUF - 1, ring, 0)

    for b in range(_NBUF):
        s = _S - _NBUF + b
        gather(s, b).wait()
        put(s, b).wait()
        transpose(b)
        put(s, b).start()

    for b in range(_NBUF):
        put(0, b).wait()


_lookup = pl.kernel(
    _lookup_body,
    mesh=_MESH,
    out_type=jax.ShapeDtypeStruct((_S, _D // 8, _CB, 8, 128), jnp.float32),
    scratch_types=[
        pltpu.VMEM((_S, 128), jnp.int32),
        pltpu.VMEM((_NBUF, 128, _D), jnp.float32),
        pltpu.VMEM((_NBUF, _D // 8, 8, 128), jnp.float32),
    ] + [pltpu.SemaphoreType.DMA] * (2 * _NBUF),
    compiler_params=pltpu.CompilerParams(
        use_tc_tiling_on_sc=False, needs_layout_passes=False),
)


def kernel(tokens, table):
    scaled_rows = _repack(table.T)
    out5 = _lookup(tokens.T.astype(jnp.int32),
                   scaled_rows.reshape(_V, _D))
    return jnp.transpose(out5, (2, 4, 0, 1, 3)).reshape(_B, _S, _D)


# speedup vs baseline: 1.7725x; 1.7725x over previous
import math

import jax
import jax.numpy as jnp
from jax import lax
from jax.experimental import pallas as pl
from jax.experimental.pallas import tpu as pltpu
from jax.experimental.pallas import tpu_sc as plsc

_V = 1_000_000
_D = 64
_L = 16
_NC, _NS = 2, 16
_NW = _NC * _NS
_B, _S = 4096, 200
_CB = _B // 128
_SCALE = math.sqrt(_D)

_MESH = plsc.VectorSubcoreMesh(core_axis_name="c", subcore_axis_name="s")


_AW = 256
_AU = _V // _AW
_AV0MAX = (_AU - 1) * _AW
_AT0 = _AU * _AW
_ATW = _V - _AT0
_AK = 124


def _repack_body(tabT, out_lin, inb0, inb1, outb0, outb1, tin, tout,
                 si0, si1, so0, so1):
    inb = (inb0, inb1)
    outb = (outb0, outb1)
    sem_in = (si0, si1)
    sem_out = (so0, so1)
    wid = lax.axis_index("s") * _NC + lax.axis_index("c")
    iota = lax.iota(jnp.int32, _L)

    def v0_of(k):
        return jnp.minimum((k * _NW + wid) * _AW, _AV0MAX)

    def rd(k, b):
        return pltpu.make_async_copy(
            tabT.at[:, pl.ds(v0_of(k), _AW)], inb[b], sem_in[b])

    def wr(k, b):
        return pltpu.make_async_copy(
            outb[b], out_lin.at[pl.ds(v0_of(k) * _D, _AW * _D)], sem_out[b])

    def transpose(b):
        for a in range(_D // _L):
            for k in range(_L):
                fvec = ((iota + k) & 15) + a * _L

                def vblk(m, carry, fvec=fvec):
                    vvec = iota + m * _L
                    vec = plsc.load_gather(inb[b], [fvec, vvec]) * _SCALE
                    plsc.store_scatter(outb[b], [(vvec << 6) + fvec], vec)
                    return carry

                lax.fori_loop(0, _AW // _L, vblk, 0)

    def step(k2, carry):
        for b in range(2):
            k = k2 * 2 + b
            rd(k, b).wait()

            @pl.when(k2 >= 1)
            def _():
                wr(k, b).wait()

            transpose(b)
            wr(k, b).start()

            @pl.when(k + 2 < _AK)
            def _():
                rd(k + 2, b).start()
        return carry

    for b in range(2):
        rd(b, b).start()

    @pl.when(wid == 0)
    def _tail():
        pltpu.sync_copy(tabT.at[:, pl.ds(_AT0, _ATW)], tin)

        def row(f, carry):
            for m in range(_ATW // _L):
                vec = tin[f, pl.ds(m * _L, _L)] * _SCALE
                plsc.store_scatter(
                    tout, [(iota + m * _L) * _D + f], vec)
            return carry
        lax.fori_loop(0, _D, row, 0)
        pltpu.sync_copy(tout, out_lin.at[pl.ds(_AT0 * _D, _ATW * _D)])

    lax.fori_loop(0, _AK // 2, step, 0)

    for b in range(2):
        wr(0, b).wait()


_repack = pl.kernel(
    _repack_body,
    mesh=_MESH,
    out_type=jax.ShapeDtypeStruct((_V * _D,), jnp.float32),
    scratch_types=[
        pltpu.VMEM((_D, _AW), jnp.float32),
        pltpu.VMEM((_D, _AW), jnp.float32),
        pltpu.VMEM((_AW * _D,), jnp.float32),
        pltpu.VMEM((_AW * _D,), jnp.float32),
        pltpu.VMEM((_D, _ATW), jnp.float32),
        pltpu.VMEM((_ATW * _D,), jnp.float32),
    ] + [pltpu.SemaphoreType.DMA] * 4,
    compiler_params=pltpu.CompilerParams(needs_layout_passes=False),
)


_NBUF = 4


def _lookup_body(tokT, table2d, out5, idx_v, *bufs):
    inb = bufs[:_NBUF]
    outb = bufs[_NBUF:2 * _NBUF]
    sem_in = bufs[2 * _NBUF:3 * _NBUF]
    sem_out = bufs[3 * _NBUF:]
    wid = lax.axis_index("s") * _NC + lax.axis_index("c")
    iota = lax.iota(jnp.int32, _L)

    pltpu.sync_copy(tokT.at[:, pl.ds(wid * 128, 128)], idx_v)

    def gather(s, b):
        return pltpu.make_async_copy(
            table2d.at[idx_v.at[s]], inb[b], sem_in[b])

    def put(s, b):
        return pltpu.make_async_copy(
            outb[b], out5.at[s, :, wid], sem_out[b])

    def transpose(b):
        for jb in range(_D // _L):
            for k in range(_L):
                dvec = ((iota + k) & 15) + jb * _L
                fbv = lax.shift_right_logical(dvec, 3)
                fiv = dvec & 7

                def tblk(tb, carry, dvec=dvec, fbv=fbv, fiv=fiv):
                    tvec = iota + tb * _L
                    vec = plsc.load_gather(inb[b], [tvec, dvec])
                    plsc.store_scatter(outb[b], [fbv, fiv, tvec], vec)
                    return carry

                lax.fori_loop(0, 128 // _L, tblk, 0)

    for b in range(_NBUF):
        gather(b, b).start()

    def ring(t, carry):
        for b in range(_NBUF):
            s = t * _NBUF + b
            gather(s, b).wait()

            @pl.when(t >= 1)
            def _():
                put(s, b).wait()

            transpose(b)
            put(s, b).start()

            @pl.when(s + _NBUF < _S)
            def _():
                gather(s + _NBUF, b).start()
        return carry

    lax.fori_loop(0, _S // _NBUF, ring, 0)

    for b in range(_NBUF):
        put(0, b).wait()


_lookup = pl.kernel(
    _lookup_body,
    mesh=_MESH,
    out_type=jax.ShapeDtypeStruct((_S, _D // 8, _CB, 8, 128), jnp.float32),
    scratch_types=[pltpu.VMEM((_S, 128), jnp.int32)]
    + [pltpu.VMEM((128, _D), jnp.float32) for _ in range(_NBUF)]
    + [pltpu.VMEM((_D // 8, 8, 128), jnp.float32) for _ in range(_NBUF)]
    + [pltpu.SemaphoreType.DMA] * (2 * _NBUF),
    compiler_params=pltpu.CompilerParams(
        use_tc_tiling_on_sc=False, needs_layout_passes=False),
)


def kernel(tokens, table):
    scaled_rows = _repack(table.T)
    out5 = _lookup(tokens.T.astype(jnp.int32),
                   scaled_rows.reshape(_V, _D))
    return jnp.transpose(out5, (2, 4, 0, 1, 3)).reshape(_B, _S, _D)


# speedup vs baseline: 4.6058x; 2.5985x over previous
import math

import jax
import jax.numpy as jnp
from jax import lax
from jax.experimental import pallas as pl
from jax.experimental.pallas import tpu as pltpu
from jax.experimental.pallas import tpu_sc as plsc

_V = 1_000_000
_D = 64
_L = 16
_NC, _NS = 2, 16
_NW = _NC * _NS
_B, _S = 4096, 200
_CB = _B // 128
_SCALE = math.sqrt(_D)

_MESH = plsc.VectorSubcoreMesh(core_axis_name="c", subcore_axis_name="s")


_AW = 256
_AU = _V // _AW
_AV0MAX = (_AU - 1) * _AW
_AT0 = _AU * _AW
_ATW = _V - _AT0
_AK = 124


def _repack_body(tabT, out_lin, inb0, inb1, outb0, outb1, tin, tout,
                 si0, si1, so0, so1):
    inb = (inb0, inb1)
    outb = (outb0, outb1)
    sem_in = (si0, si1)
    sem_out = (so0, so1)
    wid = lax.axis_index("s") * _NC + lax.axis_index("c")
    iota = lax.iota(jnp.int32, _L)

    def v0_of(k):
        return jnp.minimum((k * _NW + wid) * _AW, _AV0MAX)

    def rd(k, b):
        return pltpu.make_async_copy(
            tabT.at[:, pl.ds(v0_of(k), _AW)], inb[b], sem_in[b])

    def wr(k, b):
        return pltpu.make_async_copy(
            outb[b], out_lin.at[pl.ds(v0_of(k) * _D, _AW * _D)], sem_out[b])

    vvecs = [iota + m * _L for m in range(_AW // _L)]
    vshs = [(iota + m * _L) << 6 for m in range(_AW // _L)]

    def transpose(b):
        for a in range(_D // _L):

            @plsc.parallel_loop(0, _L, unroll=2)
            def _diag(k, a=a):
                fvec = ((iota + k) & 15) + a * _L
                for m in range(_AW // _L):
                    vec = plsc.load_gather(inb[b], [fvec, vvecs[m]]) * _SCALE
                    plsc.store_scatter(outb[b], [vshs[m] + fvec], vec)

    def step(k2, carry):
        for b in range(2):
            k = k2 * 2 + b
            rd(k, b).wait()

            @pl.when(k2 >= 1)
            def _():
                wr(k, b).wait()

            transpose(b)
            wr(k, b).start()

            @pl.when(k + 2 < _AK)
            def _():
                rd(k + 2, b).start()
        return carry

    for b in range(2):
        rd(b, b).start()

    @pl.when(wid == 0)
    def _tail():
        pltpu.sync_copy(tabT.at[:, pl.ds(_AT0, _ATW)], tin)

        def row(f, carry):
            for m in range(_ATW // _L):
                vec = tin[f, pl.ds(m * _L, _L)] * _SCALE
                plsc.store_scatter(
                    tout, [(iota + m * _L) * _D + f], vec)
            return carry
        lax.fori_loop(0, _D, row, 0)
        pltpu.sync_copy(tout, out_lin.at[pl.ds(_AT0 * _D, _ATW * _D)])

    lax.fori_loop(0, _AK // 2, step, 0)

    for b in range(2):
        wr(0, b).wait()


_repack = pl.kernel(
    _repack_body,
    mesh=_MESH,
    out_type=jax.ShapeDtypeStruct((_V * _D,), jnp.float32),
    scratch_types=[
        pltpu.VMEM((_D, _AW), jnp.float32),
        pltpu.VMEM((_D, _AW), jnp.float32),
        pltpu.VMEM((_AW * _D,), jnp.float32),
        pltpu.VMEM((_AW * _D,), jnp.float32),
        pltpu.VMEM((_D, _ATW), jnp.float32),
        pltpu.VMEM((_ATW * _D,), jnp.float32),
    ] + [pltpu.SemaphoreType.DMA] * 4,
    compiler_params=pltpu.CompilerParams(needs_layout_passes=False),
)


_NBUF = 4


def _lookup_body(tokT, table2d, out5, idx_v, *bufs):
    inb = bufs[:_NBUF]
    outb = bufs[_NBUF:2 * _NBUF]
    sem_in = bufs[2 * _NBUF:3 * _NBUF]
    sem_out = bufs[3 * _NBUF:]
    wid = lax.axis_index("s") * _NC + lax.axis_index("c")
    iota = lax.iota(jnp.int32, _L)

    pltpu.sync_copy(tokT.at[:, pl.ds(wid * 128, 128)], idx_v)

    def gather(s, b):
        return pltpu.make_async_copy(
            table2d.at[idx_v.at[s]], inb[b], sem_in[b])

    def put(s, b):
        return pltpu.make_async_copy(
            outb[b], out5.at[s, :, wid], sem_out[b])

    tvecs = [iota + tb * _L for tb in range(128 // _L)]

    def transpose(b):
        for jb in range(_D // _L):

            @plsc.parallel_loop(0, _L, unroll=2)
            def _diag(k, jb=jb):
                dvec = ((iota + k) & 15) + jb * _L
                fbv = lax.shift_right_logical(dvec, 3)
                fiv = dvec & 7
                for tb in range(128 // _L):
                    vec = plsc.load_gather(inb[b], [tvecs[tb], dvec])
                    plsc.store_scatter(outb[b], [fbv, fiv, tvecs[tb]], vec)

    for b in range(_NBUF):
        gather(b, b).start()

    def ring(t, carry):
        for b in range(_NBUF):
            s = t * _NBUF + b
            gather(s, b).wait()

            @pl.when(t >= 1)
            def _():
                put(s, b).wait()

            transpose(b)
            put(s, b).start()

            @pl.when(s + _NBUF < _S)
            def _():
                gather(s + _NBUF, b).start()
        return carry

    lax.fori_loop(0, _S // _NBUF, ring, 0)

    for b in range(_NBUF):
        put(0, b).wait()


_lookup = pl.kernel(
    _lookup_body,
    mesh=_MESH,
    out_type=jax.ShapeDtypeStruct((_S, _D // 8, _CB, 8, 128), jnp.float32),
    scratch_types=[pltpu.VMEM((_S, 128), jnp.int32)]
    + [pltpu.VMEM((128, _D), jnp.float32) for _ in range(_NBUF)]
    + [pltpu.VMEM((_D // 8, 8, 128), jnp.float32) for _ in range(_NBUF)]
    + [pltpu.SemaphoreType.DMA] * (2 * _NBUF),
    compiler_params=pltpu.CompilerParams(
        use_tc_tiling_on_sc=False, needs_layout_passes=False),
)


def kernel(tokens, table):
    scaled_rows = _repack(table.T)
    out5 = _lookup(tokens.T.astype(jnp.int32),
                   scaled_rows.reshape(_V, _D))
    return jnp.transpose(out5, (2, 4, 0, 1, 3)).reshape(_B, _S, _D)


# speedup vs baseline: 4.7249x; 1.0259x over previous
import math

import jax
import jax.numpy as jnp
from jax import lax
from jax.experimental import pallas as pl
from jax.experimental.pallas import tpu as pltpu
from jax.experimental.pallas import tpu_sc as plsc

_V = 1_000_000
_D = 64
_L = 16
_NC, _NS = 2, 16
_NW = _NC * _NS
_B, _S = 4096, 200
_CB = _B // 128
_SCALE = math.sqrt(_D)

_MESH = plsc.VectorSubcoreMesh(core_axis_name="c", subcore_axis_name="s")


_AW = 256
_AU = _V // _AW
_AV0MAX = (_AU - 1) * _AW
_AT0 = _AU * _AW
_ATW = _V - _AT0
_AK = 124


def _repack_body(tabT, out_lin, inb0, inb1, outb0, outb1, tin, tout,
                 si0, si1, so0, so1):
    inb = (inb0, inb1)
    outb = (outb0, outb1)
    sem_in = (si0, si1)
    sem_out = (so0, so1)
    wid = lax.axis_index("s") * _NC + lax.axis_index("c")
    iota = lax.iota(jnp.int32, _L)

    def v0_of(k):
        return jnp.minimum((k * _NW + wid) * _AW, _AV0MAX)

    def rd(k, b):
        return pltpu.make_async_copy(
            tabT.at[:, pl.ds(v0_of(k), _AW)], inb[b], sem_in[b])

    def wr(k, b):
        return pltpu.make_async_copy(
            outb[b], out_lin.at[pl.ds(v0_of(k) * _D, _AW * _D)], sem_out[b])

    vvecs = [iota + m * _L for m in range(_AW // _L)]
    vshs = [(iota + m * _L) << 6 for m in range(_AW // _L)]

    def transpose(b):
        for a in range(_D // _L):

            @plsc.parallel_loop(0, _L, unroll=4)
            def _diag(k, a=a):
                fvec = ((iota + k) & 15) + a * _L
                for m in range(_AW // _L):
                    vec = plsc.load_gather(inb[b], [fvec, vvecs[m]]) * _SCALE
                    plsc.store_scatter(outb[b], [vshs[m] + fvec], vec)

    def step(k2, carry):
        for b in range(2):
            k = k2 * 2 + b
            rd(k, b).wait()

            @pl.when(k2 >= 1)
            def _():
                wr(k, b).wait()

            transpose(b)
            wr(k, b).start()

            @pl.when(k + 2 < _AK)
            def _():
                rd(k + 2, b).start()
        return carry

    for b in range(2):
        rd(b, b).start()

    @pl.when(wid == 0)
    def _tail():
        pltpu.sync_copy(tabT.at[:, pl.ds(_AT0, _ATW)], tin)

        def row(f, carry):
            for m in range(_ATW // _L):
                vec = tin[f, pl.ds(m * _L, _L)] * _SCALE
                plsc.store_scatter(
                    tout, [(iota + m * _L) * _D + f], vec)
            return carry
        lax.fori_loop(0, _D, row, 0)
        pltpu.sync_copy(tout, out_lin.at[pl.ds(_AT0 * _D, _ATW * _D)])

    lax.fori_loop(0, _AK // 2, step, 0)

    for b in range(2):
        wr(0, b).wait()


_repack = pl.kernel(
    _repack_body,
    mesh=_MESH,
    out_type=jax.ShapeDtypeStruct((_V * _D,), jnp.float32),
    scratch_types=[
        pltpu.VMEM((_D, _AW), jnp.float32),
        pltpu.VMEM((_D, _AW), jnp.float32),
        pltpu.VMEM((_AW * _D,), jnp.float32),
        pltpu.VMEM((_AW * _D,), jnp.float32),
        pltpu.VMEM((_D, _ATW), jnp.float32),
        pltpu.VMEM((_ATW * _D,), jnp.float32),
    ] + [pltpu.SemaphoreType.DMA] * 4,
    compiler_params=pltpu.CompilerParams(needs_layout_passes=False),
)


_NBUF = 4


def _lookup_body(tokT, table2d, out5, idx_v, *bufs):
    inb = bufs[:_NBUF]
    outb = bufs[_NBUF:2 * _NBUF]
    sem_in = bufs[2 * _NBUF:3 * _NBUF]
    sem_out = bufs[3 * _NBUF:]
    wid = lax.axis_index("s") * _NC + lax.axis_index("c")
    iota = lax.iota(jnp.int32, _L)

    pltpu.sync_copy(tokT.at[:, pl.ds(wid * 128, 128)], idx_v)

    def gather(s, b):
        return pltpu.make_async_copy(
            table2d.at[idx_v.at[s]], inb[b], sem_in[b])

    def put(s, b):
        return pltpu.make_async_copy(
            outb[b], out5.at[s, :, wid], sem_out[b])

    tvecs = [iota + tb * _L for tb in range(128 // _L)]

    def transpose(b):
        for jb in range(_D // _L):

            @plsc.parallel_loop(0, _L, unroll=2)
            def _diag(k, jb=jb):
                dvec = ((iota + k) & 15) + jb * _L
                fbv = lax.shift_right_logical(dvec, 3)
                fiv = dvec & 7
                for tb in range(128 // _L):
                    vec = plsc.load_gather(inb[b], [tvecs[tb], dvec])
                    plsc.store_scatter(outb[b], [fbv, fiv, tvecs[tb]], vec)

    for b in range(_NBUF):
        gather(b, b).start()

    def ring(t, carry):
        for b in range(_NBUF):
            s = t * _NBUF + b
            gather(s, b).wait()

            @pl.when(t >= 1)
            def _():
                put(s, b).wait()

            transpose(b)
            put(s, b).start()

            @pl.when(s + _NBUF < _S)
            def _():
                gather(s + _NBUF, b).start()
        return carry

    lax.fori_loop(0, _S // _NBUF, ring, 0)

    for b in range(_NBUF):
        put(0, b).wait()


_lookup = pl.kernel(
    _lookup_body,
    mesh=_MESH,
    out_type=jax.ShapeDtypeStruct((_S, _D // 8, _CB, 8, 128), jnp.float32),
    scratch_types=[pltpu.VMEM((_S, 128), jnp.int32)]
    + [pltpu.VMEM((128, _D), jnp.float32) for _ in range(_NBUF)]
    + [pltpu.VMEM((_D // 8, 8, 128), jnp.float32) for _ in range(_NBUF)]
    + [pltpu.SemaphoreType.DMA] * (2 * _NBUF),
    compiler_params=pltpu.CompilerParams(
        use_tc_tiling_on_sc=False, needs_layout_passes=False),
)


def kernel(tokens, table):
    scaled_rows = _repack(table.T)
    out5 = _lookup(tokens.T.astype(jnp.int32),
                   scaled_rows.reshape(_V, _D))
    return jnp.transpose(out5, (2, 4, 0, 1, 3)).reshape(_B, _S, _D)
